# baseline (speedup 1.0000x reference)
import functools

import jax
import jax.numpy as jnp
from jax import lax
from jax.experimental import pallas as pl
from jax.experimental.pallas import tpu as pltpu
from jax.experimental.pallas import tpu_sc as plsc

_LANES = 16


def _sc_gather_add(x_flat, idx, table):
    B, D = x_flat.shape
    info = plsc.get_sparse_core_info()
    NC, NS = info.num_cores, info.num_subcores
    NW = NC * NS
    b_per_w = B // NW
    KG = 16
    KH = 8
    n_chunks = b_per_w // KG
    NV = D // _LANES

    mesh = plsc.VectorSubcoreMesh(core_axis_name="c", subcore_axis_name="s")

    @functools.partial(
        pl.kernel,
        mesh=mesh,
        out_type=jax.ShapeDtypeStruct((B, D), jnp.float32),
        scratch_types=[
            pltpu.VMEM((b_per_w,), jnp.int32),
            pltpu.VMEM((KG, D), jnp.float32),
            pltpu.VMEM((KG, D), jnp.float32),
            pltpu.VMEM((KH, D), jnp.float32),
            pltpu.VMEM((KH, D), jnp.float32),
            pltpu.SemaphoreType.DMA,
            pltpu.SemaphoreType.DMA,
            pltpu.SemaphoreType.DMA,
            pltpu.SemaphoreType.DMA,
            pltpu.SemaphoreType.DMA,
            pltpu.SemaphoreType.DMA,
            pltpu.SemaphoreType.DMA,
            pltpu.SemaphoreType.DMA,
            pltpu.SemaphoreType.DMA,
        ],
    )
    def gather_add(x_hbm, idx_hbm, table_hbm, out_hbm, idx_v,
                   gbuf0, gbuf1, xbuf0, xbuf1,
                   gsem0, gsem1, xsem0, xsem1,
                   ssem00, ssem01, ssem10, ssem11, isem):
        gbufs, xbufs = (gbuf0, gbuf1), (xbuf0, xbuf1)
        gsems, xsems = (gsem0, gsem1), (xsem0, xsem1)
        ssems = ((ssem00, ssem01), (ssem10, ssem11))

        wid = lax.axis_index("s") * NC + lax.axis_index("c")
        base = wid * b_per_w
        idx_dma = pltpu.async_copy(
            idx_hbm.at[pl.ds(base, b_per_w)], idx_v, isem)

        def issue_x(c, h):
            pltpu.async_copy(
                x_hbm.at[pl.ds(base + c * KG + h * KH, KH)],
                xbufs[h], xsems[h])

        def wait_x(h):
            pltpu.make_async_copy(
                x_hbm.at[pl.ds(0, KH)], xbufs[h], xsems[h]).wait()

        def issue_g(c, b):
            pltpu.async_copy(
                table_hbm.at[idx_v.at[pl.ds(c * KG, KG)]], gbufs[b], gsems[b])

        def wait_g(b):
            pltpu.make_async_copy(
                table_hbm.at[idx_v.at[pl.ds(0, KG)]], gbufs[b], gsems[b]).wait()

        def issue_s(c, b, h):
            pltpu.async_copy(
                gbufs[b].at[pl.ds(h * KH, KH)],
                out_hbm.at[pl.ds(base + c * KG + h * KH, KH)], ssems[b][h])

        def wait_s(b, h):
            pltpu.make_async_copy(
                gbufs[b].at[pl.ds(h * KH, KH)],
                out_hbm.at[pl.ds(0, KH)], ssems[b][h]).wait()

        issue_x(0, 0)
        issue_x(0, 1)
        idx_dma.wait()
        issue_g(0, 0)
        issue_g(1, 1)

        def pair_body(c2, carry):
            for b in (0, 1):
                c = 2 * c2 + b
                wait_g(b)
                for h in (0, 1):
                    wait_x(h)

                    def row_body(r, rc):
                        for j in range(NV):
                            sl = pl.ds(j * _LANES, _LANES)
                            plsc.addupdate(
                                gbufs[b].at[h * KH + r, sl], xbufs[h][r, sl])
                        return rc

                    lax.fori_loop(0, KH, row_body, 0)

                    @pl.when(c + 1 < n_chunks)
                    def _prefetch_x():
                        issue_x(c + 1, h)

                    issue_s(c, b, h)

                @pl.when(c + 2 < n_chunks)
                def _prefetch_g():
                    wait_s(b, 0)
                    wait_s(b, 1)
                    issue_g(c + 2, b)
            return carry

        lax.fori_loop(0, n_chunks // 2, pair_body, 0)
        wait_s(0, 0)
        wait_s(0, 1)
        wait_s(1, 0)
        wait_s(1, 1)

    return gather_add(x_flat, idx, table)


def kernel(x, time_idx, emb_table):
    Bb, T, D = x.shape
    if T == time_idx.shape[1]:
        return x
    idx = time_idx[:, :T].reshape(-1).astype(jnp.int32)
    x_flat = x.reshape(Bb * T, D)
    out = _sc_gather_add(x_flat, idx, emb_table)
    return out.reshape(Bb, T, D)
